# Initial kernel scaffold; baseline (speedup 1.0000x reference)
import jax
import jax.numpy as jnp
from jax.experimental import pallas as pl


def kernel(x, router_w, router_b, W1, b1, W2, b2, ln_g, ln_b):
    raise NotImplementedError("write your pallas kernel here")


# speedup vs baseline: 11.5059x; 11.5059x over previous
import functools
import math

import jax
import jax.numpy as jnp
from jax.experimental import pallas as pl
from jax.experimental.pallas import tpu as pltpu

B = 2
S = 2048
D_MODEL = 768
D_FF = 2048
E = 64
CAP_FACTOR = 1.25
C = int(math.ceil(B * S * CAP_FACTOR / E))


def _router_kernel(x_ref, rw_ref, rb_ref, out_ref):
    pooled = jnp.mean(x_ref[...], axis=1)
    logits = jnp.dot(pooled, rw_ref[...],
                     preferred_element_type=jnp.float32) + rb_ref[...]
    maxv = jnp.max(logits, axis=1, keepdims=True)
    idx = jax.lax.broadcasted_iota(jnp.int32, (B, E), 1)
    masked = jnp.where(logits >= maxv, idx, jnp.int32(E))
    am = jnp.min(masked, axis=1, keepdims=True)
    out_ref[...] = jnp.broadcast_to(am, (B, 128))


def _moe_kernel(e_ref, x_ref, w1_ref, b1_ref, w2_ref, b2_ref, g_ref, bb_ref,
                o_ref):
    b = pl.program_id(0)
    xx = x_ref[0]
    xc = xx[:C]
    h = jnp.maximum(
        jnp.dot(xc, w1_ref[0], preferred_element_type=jnp.float32)
        + b1_ref[0], 0.0)
    y = (jnp.dot(h, w2_ref[0], preferred_element_type=jnp.float32)
         + b2_ref[0])
    valid = jnp.logical_or(b == 0, e_ref[0] != e_ref[1])
    y = jnp.where(valid, y, 0.0)

    g = g_ref[...]
    bb = bb_ref[...]

    res_c = xc + y
    mu_c = jnp.mean(res_c, axis=1, keepdims=True)
    var_c = jnp.mean((res_c - mu_c) ** 2, axis=1, keepdims=True)
    o_ref[0, :C, :] = (res_c - mu_c) * jax.lax.rsqrt(var_c + 1e-5) * g + bb

    rest = xx[C:]
    mu_r = jnp.mean(rest, axis=1, keepdims=True)
    var_r = jnp.mean((rest - mu_r) ** 2, axis=1, keepdims=True)
    o_ref[0, C:, :] = (rest - mu_r) * jax.lax.rsqrt(var_r + 1e-5) * g + bb


@functools.partial(jax.jit, static_argnames=("interpret",))
def _run(x, router_w, router_b, W1, b1, W2, b2, ln_g, ln_b, interpret=False):
    rb2 = router_b.reshape(1, E)
    g2 = ln_g.reshape(1, D_MODEL)
    lb2 = ln_b.reshape(1, D_MODEL)
    b1r = b1.reshape(E, 1, D_FF)
    b2r = b2.reshape(E, 1, D_MODEL)

    e_out = pl.pallas_call(
        _router_kernel,
        out_shape=jax.ShapeDtypeStruct((B, 128), jnp.int32),
        interpret=interpret,
    )(x, router_w, rb2)
    e_idx = e_out[:, 0]

    grid_spec = pltpu.PrefetchScalarGridSpec(
        num_scalar_prefetch=1,
        grid=(B,),
        in_specs=[
            pl.BlockSpec((1, S, D_MODEL), lambda b, e: (b, 0, 0)),
            pl.BlockSpec((1, D_MODEL, D_FF), lambda b, e: (e[b], 0, 0)),
            pl.BlockSpec((1, 1, D_FF), lambda b, e: (e[b], 0, 0)),
            pl.BlockSpec((1, D_FF, D_MODEL), lambda b, e: (e[b], 0, 0)),
            pl.BlockSpec((1, 1, D_MODEL), lambda b, e: (e[b], 0, 0)),
            pl.BlockSpec((1, D_MODEL), lambda b, e: (0, 0)),
            pl.BlockSpec((1, D_MODEL), lambda b, e: (0, 0)),
        ],
        out_specs=pl.BlockSpec((1, S, D_MODEL), lambda b, e: (b, 0, 0)),
    )
    out = pl.pallas_call(
        _moe_kernel,
        grid_spec=grid_spec,
        out_shape=jax.ShapeDtypeStruct((B, S, D_MODEL), jnp.float32),
        interpret=interpret,
    )(e_idx, x, W1, b1r, W2, b2r, g2, lb2)
    return out


def kernel(x, router_w, router_b, W1, b1, W2, b2, ln_g, ln_b):
    return _run(x, router_w, router_b, W1, b1, W2, b2, ln_g, ln_b)


# speedup vs baseline: 12.9330x; 1.1240x over previous
import functools
import math

import jax
import jax.numpy as jnp
from jax.experimental import pallas as pl
from jax.experimental.pallas import tpu as pltpu

B = 2
S = 2048
D_MODEL = 768
D_FF = 2048
E = 64
CAP_FACTOR = 1.25
C = int(math.ceil(B * S * CAP_FACTOR / E))


def _router_ln_kernel(x_ref, rw_ref, rb_ref, g_ref, bb_ref, e_ref, o_ref):
    xx = x_ref[0]
    pooled = jnp.mean(xx, axis=0, keepdims=True)
    logits = jnp.dot(pooled, rw_ref[...],
                     preferred_element_type=jnp.float32) + rb_ref[...]
    maxv = jnp.max(logits, axis=1, keepdims=True)
    idx = jax.lax.broadcasted_iota(jnp.int32, (1, E), 1)
    masked = jnp.where(logits >= maxv, idx, jnp.int32(E))
    am = jnp.min(masked, axis=1, keepdims=True)
    e_ref[0] = jnp.broadcast_to(am, (8, 128))
    mu = jnp.mean(xx, axis=1, keepdims=True)
    var = jnp.mean((xx - mu) ** 2, axis=1, keepdims=True)
    o_ref[0] = (xx - mu) * jax.lax.rsqrt(var + 1e-5) * g_ref[...] + bb_ref[...]


def _expert_kernel(e_ref, x_ref, w1_ref, b1_ref, w2_ref, b2_ref, g_ref,
                   bb_ref, prev_ref, o_ref):
    del prev_ref
    b = pl.program_id(0)
    xc = x_ref[0]
    h = jnp.maximum(
        jnp.dot(xc, w1_ref[0], preferred_element_type=jnp.float32)
        + b1_ref[0], 0.0)
    y = (jnp.dot(h, w2_ref[0], preferred_element_type=jnp.float32)
         + b2_ref[0])
    valid = jnp.logical_or(b == 0, e_ref[0] != e_ref[1])
    res = xc + jnp.where(valid, y, 0.0)
    mu = jnp.mean(res, axis=1, keepdims=True)
    var = jnp.mean((res - mu) ** 2, axis=1, keepdims=True)
    o_ref[0] = (res - mu) * jax.lax.rsqrt(var + 1e-5) * g_ref[...] + bb_ref[...]


@functools.partial(jax.jit, static_argnames=("interpret",))
def _run(x, router_w, router_b, W1, b1, W2, b2, ln_g, ln_b, interpret=False):
    rb2 = router_b.reshape(1, E)
    g2 = ln_g.reshape(1, D_MODEL)
    lb2 = ln_b.reshape(1, D_MODEL)
    b1r = b1.reshape(E, 1, D_FF)
    b2r = b2.reshape(E, 1, D_MODEL)

    e_out, out_a = pl.pallas_call(
        _router_ln_kernel,
        grid=(B,),
        in_specs=[
            pl.BlockSpec((1, S, D_MODEL), lambda b: (b, 0, 0)),
            pl.BlockSpec((D_MODEL, E), lambda b: (0, 0)),
            pl.BlockSpec((1, E), lambda b: (0, 0)),
            pl.BlockSpec((1, D_MODEL), lambda b: (0, 0)),
            pl.BlockSpec((1, D_MODEL), lambda b: (0, 0)),
        ],
        out_specs=[
            pl.BlockSpec((1, 8, 128), lambda b: (b, 0, 0)),
            pl.BlockSpec((1, S, D_MODEL), lambda b: (b, 0, 0)),
        ],
        out_shape=[
            jax.ShapeDtypeStruct((B, 8, 128), jnp.int32),
            jax.ShapeDtypeStruct((B, S, D_MODEL), jnp.float32),
        ],
        interpret=interpret,
    )(x, router_w, rb2, g2, lb2)
    e_idx = e_out[:, 0, 0]

    grid_spec = pltpu.PrefetchScalarGridSpec(
        num_scalar_prefetch=1,
        grid=(B,),
        in_specs=[
            pl.BlockSpec((1, C, D_MODEL), lambda b, e: (b, 0, 0)),
            pl.BlockSpec((1, D_MODEL, D_FF), lambda b, e: (e[b], 0, 0)),
            pl.BlockSpec((1, 1, D_FF), lambda b, e: (e[b], 0, 0)),
            pl.BlockSpec((1, D_FF, D_MODEL), lambda b, e: (e[b], 0, 0)),
            pl.BlockSpec((1, 1, D_MODEL), lambda b, e: (e[b], 0, 0)),
            pl.BlockSpec((1, D_MODEL), lambda b, e: (0, 0)),
            pl.BlockSpec((1, D_MODEL), lambda b, e: (0, 0)),
            pl.BlockSpec((1, C, D_MODEL), lambda b, e: (b, 0, 0)),
        ],
        out_specs=pl.BlockSpec((1, C, D_MODEL), lambda b, e: (b, 0, 0)),
    )
    out = pl.pallas_call(
        _expert_kernel,
        grid_spec=grid_spec,
        out_shape=jax.ShapeDtypeStruct((B, S, D_MODEL), jnp.float32),
        input_output_aliases={8: 0},
        interpret=interpret,
    )(e_idx, x, W1, b1r, W2, b2r, g2, lb2, out_a)
    return out


def kernel(x, router_w, router_b, W1, b1, W2, b2, ln_g, ln_b):
    return _run(x, router_w, router_b, W1, b1, W2, b2, ln_g, ln_b)
